# initial kernel scaffold (unmeasured)
import jax
import jax.numpy as jnp
from jax import lax
from jax.experimental import pallas as pl
from jax.experimental.pallas import tpu as pltpu


def kernel(
    x,
):
    def body(*refs):
        pass

    out_shape = jax.ShapeDtypeStruct(..., jnp.float32)
    return pl.pallas_call(body, out_shape=out_shape)(...)


# baseline (device time: 20555 ns/iter reference)
import jax
import jax.numpy as jnp
from jax import lax
from jax.experimental import pallas as pl
from jax.experimental.pallas import tpu as pltpu

N_DEV = 32
BIG = 1e9


def kernel(x):
    m_per, n = x.shape

    def body(x_ref, out_ref, stage_ref, comm_ref, send_sems, recv_sems):
        my = lax.axis_index("i")

        xv = x_ref[:, :]
        local_max = jnp.max(xv, axis=0, keepdims=True)
        rows = lax.broadcasted_iota(jnp.int32, (m_per, n), 0).astype(jnp.float32)
        cand = jnp.where(xv == local_max, rows, jnp.float32(BIG))
        local_arg = jnp.min(cand, axis=0, keepdims=True)
        g_arg = local_arg + my.astype(jnp.float32) * m_per

        stage_ref[0:1, :] = local_max
        stage_ref[1:2, :] = g_arg
        comm_ref[pl.ds(my, 1)] = jnp.reshape(stage_ref[:, :], (1, 2, n))

        send_rdmas = []
        for k in range(1, N_DEV):
            dst = (my + k) % N_DEV
            rdma = pltpu.make_async_remote_copy(
                src_ref=stage_ref,
                dst_ref=comm_ref.at[my],
                send_sem=send_sems.at[k - 1],
                recv_sem=recv_sems.at[my],
                device_id=(dst,),
                device_id_type=pl.DeviceIdType.MESH,
            )
            rdma.start()
            send_rdmas.append(rdma)

        for k in range(1, N_DEV):
            src = (my + k) % N_DEV
            recv = pltpu.make_async_remote_copy(
                src_ref=stage_ref,
                dst_ref=comm_ref.at[src],
                send_sem=send_sems.at[0],
                recv_sem=recv_sems.at[src],
                device_id=(src,),
                device_id_type=pl.DeviceIdType.MESH,
            )
            recv.wait_recv()

        vals = comm_ref[:, 0, :]
        idxs = comm_ref[:, 1, :]
        gmax = jnp.max(vals, axis=0, keepdims=True)
        cand2 = jnp.where(vals == gmax, idxs, jnp.float32(BIG))
        gidx = jnp.min(cand2, axis=0, keepdims=True)
        out_ref[0:1, :] = gmax
        out_ref[1:2, :] = gidx

        for rdma in send_rdmas:
            rdma.wait_send()

    return pl.pallas_call(
        body,
        out_shape=jax.ShapeDtypeStruct((2, n), jnp.float32),
        in_specs=[pl.BlockSpec(memory_space=pltpu.VMEM)],
        out_specs=pl.BlockSpec(memory_space=pltpu.VMEM),
        scratch_shapes=[
            pltpu.VMEM((2, n), jnp.float32),
            pltpu.VMEM((N_DEV, 2, n), jnp.float32),
            pltpu.SemaphoreType.DMA((N_DEV - 1,)),
            pltpu.SemaphoreType.DMA((N_DEV,)),
        ],
    )(x)


# device time: 13335 ns/iter; 1.5414x vs baseline; 1.5414x over previous
import jax
import jax.numpy as jnp
from jax import lax
from jax.experimental import pallas as pl
from jax.experimental.pallas import tpu as pltpu

N_DEV = 32
PLANE = 8
NZ = 4
BIG = 1e9


def kernel(x):
    m_per, n = x.shape

    def body(
        x_ref,
        out_ref,
        stage_ref,
        stage2_ref,
        plane_ref,
        zcol_ref,
        psend_sems,
        precv_sems,
        zsend_sems,
        zrecv_sems,
    ):
        my = lax.axis_index("i")
        z_g = my // PLANE
        j = my % PLANE

        barrier_sem = pltpu.get_barrier_semaphore()
        for jj in range(PLANE):
            @pl.when(jj != j)
            def _():
                pl.semaphore_signal(
                    barrier_sem, inc=1,
                    device_id=(z_g * PLANE + jj,),
                    device_id_type=pl.DeviceIdType.MESH,
                )
        for zz in range(NZ):
            @pl.when(zz != z_g)
            def _():
                pl.semaphore_signal(
                    barrier_sem, inc=1,
                    device_id=(zz * PLANE + j,),
                    device_id_type=pl.DeviceIdType.MESH,
                )

        xv = x_ref[:, :]
        local_max = jnp.max(xv, axis=0, keepdims=True)
        rows = lax.broadcasted_iota(jnp.int32, (m_per, n), 0).astype(jnp.float32)
        cand = jnp.where(xv == local_max, rows, jnp.float32(BIG))
        local_arg = jnp.min(cand, axis=0, keepdims=True)
        stage_ref[0:1, :] = local_max
        stage_ref[1:2, :] = local_arg + my.astype(jnp.float32) * m_per
        plane_ref[pl.ds(j, 1)] = jnp.reshape(stage_ref[:, :], (1, 2, n))

        pl.semaphore_wait(barrier_sem, PLANE - 1 + NZ - 1)

        p_rdmas = []
        for jj in range(PLANE):
            rdma = pltpu.make_async_remote_copy(
                src_ref=stage_ref,
                dst_ref=plane_ref.at[j],
                send_sem=psend_sems.at[jj],
                recv_sem=precv_sems.at[j],
                device_id=(z_g * PLANE + jj,),
                device_id_type=pl.DeviceIdType.MESH,
            )
            p_rdmas.append((jj, rdma))

            @pl.when(jj != j)
            def _(rdma=rdma):
                rdma.start()

        for jj in range(PLANE):
            recv = pltpu.make_async_remote_copy(
                src_ref=stage_ref,
                dst_ref=plane_ref.at[jj],
                send_sem=psend_sems.at[0],
                recv_sem=precv_sems.at[jj],
                device_id=(0,),
                device_id_type=pl.DeviceIdType.MESH,
            )

            @pl.when(jj != j)
            def _(recv=recv):
                recv.wait_recv()

        pvals = plane_ref[:, 0, :]
        pidxs = plane_ref[:, 1, :]
        pmax = jnp.max(pvals, axis=0, keepdims=True)
        pcand = jnp.where(pvals == pmax, pidxs, jnp.float32(BIG))
        pidx = jnp.min(pcand, axis=0, keepdims=True)
        stage2_ref[0:1, :] = pmax
        stage2_ref[1:2, :] = pidx
        zcol_ref[pl.ds(z_g, 1)] = jnp.reshape(stage2_ref[:, :], (1, 2, n))

        z_rdmas = []
        for zz in range(NZ):
            rdma = pltpu.make_async_remote_copy(
                src_ref=stage2_ref,
                dst_ref=zcol_ref.at[z_g],
                send_sem=zsend_sems.at[zz],
                recv_sem=zrecv_sems.at[z_g],
                device_id=(zz * PLANE + j,),
                device_id_type=pl.DeviceIdType.MESH,
            )
            z_rdmas.append((zz, rdma))

            @pl.when(zz != z_g)
            def _(rdma=rdma):
                rdma.start()

        for zz in range(NZ):
            recv = pltpu.make_async_remote_copy(
                src_ref=stage2_ref,
                dst_ref=zcol_ref.at[zz],
                send_sem=zsend_sems.at[0],
                recv_sem=zrecv_sems.at[zz],
                device_id=(0,),
                device_id_type=pl.DeviceIdType.MESH,
            )

            @pl.when(zz != z_g)
            def _(recv=recv):
                recv.wait_recv()

        zvals = zcol_ref[:, 0, :]
        zidxs = zcol_ref[:, 1, :]
        gmax = jnp.max(zvals, axis=0, keepdims=True)
        zcand = jnp.where(zvals == gmax, zidxs, jnp.float32(BIG))
        gidx = jnp.min(zcand, axis=0, keepdims=True)
        out_ref[0:1, :] = gmax
        out_ref[1:2, :] = gidx

        for jj, rdma in p_rdmas:
            @pl.when(jj != j)
            def _(rdma=rdma):
                rdma.wait_send()
        for zz, rdma in z_rdmas:
            @pl.when(zz != z_g)
            def _(rdma=rdma):
                rdma.wait_send()

    return pl.pallas_call(
        body,
        out_shape=jax.ShapeDtypeStruct((2, n), jnp.float32),
        in_specs=[pl.BlockSpec(memory_space=pltpu.VMEM)],
        out_specs=pl.BlockSpec(memory_space=pltpu.VMEM),
        scratch_shapes=[
            pltpu.VMEM((2, n), jnp.float32),
            pltpu.VMEM((2, n), jnp.float32),
            pltpu.VMEM((PLANE, 2, n), jnp.float32),
            pltpu.VMEM((NZ, 2, n), jnp.float32),
            pltpu.SemaphoreType.DMA((PLANE,)),
            pltpu.SemaphoreType.DMA((PLANE,)),
            pltpu.SemaphoreType.DMA((NZ,)),
            pltpu.SemaphoreType.DMA((NZ,)),
        ],
        compiler_params=pltpu.CompilerParams(collective_id=0),
    )(x)


# device time: 13304 ns/iter; 1.5450x vs baseline; 1.0023x over previous
import jax
import jax.numpy as jnp
from jax import lax
from jax.experimental import pallas as pl
from jax.experimental.pallas import tpu as pltpu

N_DEV = 32
PLANE = 8
NZ = 4
BIG = 1e9


def kernel(x):
    m_per, n = x.shape

    def body(
        x_ref,
        out_ref,
        stage_ref,
        stage2_ref,
        plane_ref,
        zcol_ref,
        psend_sems,
        precv_sems,
        zsend_sems,
        zrecv_sems,
    ):
        my = lax.axis_index("i")
        z_g = my // PLANE
        j = my % PLANE

        barrier_sem = pltpu.get_barrier_semaphore()
        for jj in range(PLANE):
            @pl.when(jj != j)
            def _():
                pl.semaphore_signal(
                    barrier_sem, inc=1,
                    device_id=(z_g * PLANE + jj,),
                    device_id_type=pl.DeviceIdType.MESH,
                )
        for zz in range(NZ):
            @pl.when(zz != z_g)
            def _():
                pl.semaphore_signal(
                    barrier_sem, inc=1,
                    device_id=(zz * PLANE + j,),
                    device_id_type=pl.DeviceIdType.MESH,
                )

        xv = x_ref[:, :]
        local_max = jnp.max(xv, axis=0, keepdims=True)
        rows = lax.broadcasted_iota(jnp.int32, (m_per, n), 0).astype(jnp.float32)
        cand = jnp.where(xv == local_max, rows, jnp.float32(BIG))
        local_arg = jnp.min(cand, axis=0, keepdims=True)
        stage_ref[0:1, :] = local_max
        stage_ref[1:2, :] = local_arg + my.astype(jnp.float32) * m_per

        pl.semaphore_wait(barrier_sem, PLANE - 1 + NZ - 1)

        p_rdmas = []
        for jj in range(PLANE):
            rdma = pltpu.make_async_remote_copy(
                src_ref=stage_ref,
                dst_ref=plane_ref.at[j],
                send_sem=psend_sems.at[jj],
                recv_sem=precv_sems.at[j],
                device_id=(z_g * PLANE + jj,),
                device_id_type=pl.DeviceIdType.MESH,
            )
            p_rdmas.append((jj, rdma))

            @pl.when(jj != j)
            def _(rdma=rdma):
                rdma.start()

        plane_ref[pl.ds(j, 1)] = jnp.reshape(stage_ref[:, :], (1, 2, n))

        for jj in range(PLANE):
            recv = pltpu.make_async_remote_copy(
                src_ref=stage_ref,
                dst_ref=plane_ref.at[jj],
                send_sem=psend_sems.at[0],
                recv_sem=precv_sems.at[jj],
                device_id=(0,),
                device_id_type=pl.DeviceIdType.MESH,
            )

            @pl.when(jj != j)
            def _(recv=recv):
                recv.wait_recv()

        pvals = plane_ref[:, 0, :]
        pidxs = plane_ref[:, 1, :]
        pmax = jnp.max(pvals, axis=0, keepdims=True)
        pcand = jnp.where(pvals == pmax, pidxs, jnp.float32(BIG))
        pidx = jnp.min(pcand, axis=0, keepdims=True)
        stage2_ref[0:1, :] = pmax
        stage2_ref[1:2, :] = pidx

        z_rdmas = []
        for zz in range(NZ):
            rdma = pltpu.make_async_remote_copy(
                src_ref=stage2_ref,
                dst_ref=zcol_ref.at[z_g],
                send_sem=zsend_sems.at[zz],
                recv_sem=zrecv_sems.at[z_g],
                device_id=(zz * PLANE + j,),
                device_id_type=pl.DeviceIdType.MESH,
            )
            z_rdmas.append((zz, rdma))

            @pl.when(zz != z_g)
            def _(rdma=rdma):
                rdma.start()

        zcol_ref[pl.ds(z_g, 1)] = jnp.reshape(stage2_ref[:, :], (1, 2, n))

        for zz in range(NZ):
            recv = pltpu.make_async_remote_copy(
                src_ref=stage2_ref,
                dst_ref=zcol_ref.at[zz],
                send_sem=zsend_sems.at[0],
                recv_sem=zrecv_sems.at[zz],
                device_id=(0,),
                device_id_type=pl.DeviceIdType.MESH,
            )

            @pl.when(zz != z_g)
            def _(recv=recv):
                recv.wait_recv()

        zvals = zcol_ref[:, 0, :]
        zidxs = zcol_ref[:, 1, :]
        gmax = jnp.max(zvals, axis=0, keepdims=True)
        zcand = jnp.where(zvals == gmax, zidxs, jnp.float32(BIG))
        gidx = jnp.min(zcand, axis=0, keepdims=True)
        out_ref[0:1, :] = gmax
        out_ref[1:2, :] = gidx

        for jj, rdma in p_rdmas:
            @pl.when(jj != j)
            def _(rdma=rdma):
                rdma.wait_send()
        for zz, rdma in z_rdmas:
            @pl.when(zz != z_g)
            def _(rdma=rdma):
                rdma.wait_send()

    return pl.pallas_call(
        body,
        out_shape=jax.ShapeDtypeStruct((2, n), jnp.float32),
        in_specs=[pl.BlockSpec(memory_space=pltpu.VMEM)],
        out_specs=pl.BlockSpec(memory_space=pltpu.VMEM),
        scratch_shapes=[
            pltpu.VMEM((2, n), jnp.float32),
            pltpu.VMEM((2, n), jnp.float32),
            pltpu.VMEM((PLANE, 2, n), jnp.float32),
            pltpu.VMEM((NZ, 2, n), jnp.float32),
            pltpu.SemaphoreType.DMA((PLANE,)),
            pltpu.SemaphoreType.DMA((PLANE,)),
            pltpu.SemaphoreType.DMA((NZ,)),
            pltpu.SemaphoreType.DMA((NZ,)),
        ],
        compiler_params=pltpu.CompilerParams(collective_id=0),
    )(x)
